# Initial kernel scaffold; baseline (speedup 1.0000x reference)
import jax
import jax.numpy as jnp
from jax.experimental import pallas as pl


def kernel(feat0, feat1, image_shape_t):
    raise NotImplementedError("write your pallas kernel here")


# speedup vs baseline: 17.5677x; 17.5677x over previous
import numpy as np

import jax
import jax.numpy as jnp
from jax import lax
from jax.experimental import pallas as pl

NUM_CLASSES = 80
MAX_BOXES = 20
SCORE_THR = 0.6
IOU_THR = 0.5

_G0, _G1 = 19, 38
_N0 = _G0 * _G0 * 3
_N1 = _G1 * _G1 * 3
_N = _N0 + _N1
_NP = 43 * 128

_ANCHORS = np.array([[10.0, 14.0], [23.0, 27.0], [37.0, 58.0],
                     [81.0, 82.0], [135.0, 169.0], [344.0, 319.0]],
                    dtype=np.float32)
_MASK0 = [3, 4, 5]
_MASK1 = [1, 2, 3]

_INPUT = np.float32(_G0 * 32)
_IMG = np.array([720.0, 1280.0], dtype=np.float32)
_SC = np.float32(min(np.float32(_INPUT / _IMG[0]), np.float32(_INPUT / _IMG[1])))
_NEW = np.round(_IMG * _SC).astype(np.float32)
_OFF = ((_INPUT - _NEW) / np.float32(2.0) / _INPUT).astype(np.float32)
_SCALE = (_INPUT / _NEW).astype(np.float32)


def _make_consts():
    gx = np.zeros(_NP, np.float32)
    gy = np.zeros(_NP, np.float32)
    gd = np.ones(_NP, np.float32)
    aw = np.zeros(_NP, np.float32)
    ah = np.zeros(_NP, np.float32)
    n = np.arange(_N0)
    cell, a = n // 3, n % 3
    gx[:_N0] = (cell % _G0).astype(np.float32)
    gy[:_N0] = (cell // _G0).astype(np.float32)
    gd[:_N0] = float(_G0)
    anc = _ANCHORS[_MASK0][a]
    aw[:_N0] = anc[:, 0]
    ah[:_N0] = anc[:, 1]
    n = np.arange(_N1)
    cell, a = n // 3, n % 3
    gx[_N0:_N] = (cell % _G1).astype(np.float32)
    gy[_N0:_N] = (cell // _G1).astype(np.float32)
    gd[_N0:_N] = float(_G1)
    anc = _ANCHORS[_MASK1][a]
    aw[_N0:_N] = anc[:, 0]
    ah[_N0:_N] = anc[:, 1]
    return np.stack([gx, gy, gd, aw, ah], axis=0)


_CONSTS = _make_consts()


def _nms_body(t_ref, c_ref, s_out, y1_out, x1_out, y2_out, x2_out):
    def sig(x):
        return 1.0 / (1.0 + jnp.exp(-x))

    tx = t_ref[0:1, :]
    ty = t_ref[1:2, :]
    tw = t_ref[2:3, :]
    th = t_ref[3:4, :]
    tc = t_ref[4:5, :]
    tp = t_ref[5:85, :]
    gx = c_ref[0:1, :]
    gy = c_ref[1:2, :]
    gd = c_ref[2:3, :]
    aw = c_ref[3:4, :]
    ah = c_ref[4:5, :]

    bx = (sig(tx) + gx) / gd
    by = (sig(ty) + gy) / gd
    bw = jnp.exp(tw) * aw / _INPUT
    bh = jnp.exp(th) * ah / _INPUT
    yy = (by - _OFF[0]) * _SCALE[0]
    xx = (bx - _OFF[1]) * _SCALE[1]
    hh = bh * _SCALE[0]
    ww = bw * _SCALE[1]
    y1 = (yy - hh / 2.0) * _IMG[0]
    x1 = (xx - ww / 2.0) * _IMG[1]
    y2 = (yy + hh / 2.0) * _IMG[0]
    x2 = (xx + ww / 2.0) * _IMG[1]
    area = (y2 - y1) * (x2 - x1)

    s = sig(tc) * sig(tp)
    s = jnp.where(s >= SCORE_THR, s, 0.0)

    iota_n = lax.broadcasted_iota(jnp.int32, (NUM_CLASSES, _NP), 1)
    col20 = lax.broadcasted_iota(jnp.int32, (NUM_CLASSES, MAX_BOXES), 1)
    sel_s = jnp.zeros((NUM_CLASSES, MAX_BOXES), jnp.float32)
    sel_y1 = jnp.zeros((NUM_CLASSES, MAX_BOXES), jnp.float32)
    sel_x1 = jnp.zeros((NUM_CLASSES, MAX_BOXES), jnp.float32)
    sel_y2 = jnp.zeros((NUM_CLASSES, MAX_BOXES), jnp.float32)
    sel_x2 = jnp.zeros((NUM_CLASSES, MAX_BOXES), jnp.float32)

    for i in range(MAX_BOXES):
        m = jnp.max(s, axis=1, keepdims=True)
        keep = m > 0.0
        j = jnp.min(jnp.where(s == m, iota_n, _NP), axis=1, keepdims=True)
        onehot = iota_n == j
        ohf = onehot.astype(jnp.float32)
        cy1 = jnp.sum(ohf * y1, axis=1, keepdims=True)
        cx1 = jnp.sum(ohf * x1, axis=1, keepdims=True)
        cy2 = jnp.sum(ohf * y2, axis=1, keepdims=True)
        cx2 = jnp.sum(ohf * x2, axis=1, keepdims=True)
        carea = jnp.sum(ohf * area, axis=1, keepdims=True)
        iy1 = jnp.maximum(cy1, y1)
        ix1 = jnp.maximum(cx1, x1)
        iy2 = jnp.minimum(cy2, y2)
        ix2 = jnp.minimum(cx2, x2)
        inter = jnp.maximum(iy2 - iy1, 0.0) * jnp.maximum(ix2 - ix1, 0.0)
        iou = inter / (carea + area - inter + 1e-9)
        suppress = (iou > IOU_THR) | onehot
        s = jnp.where(keep & suppress, 0.0, s)
        at_i = col20 == i
        sel_s = jnp.where(at_i, jnp.where(keep, m, 0.0), sel_s)
        sel_y1 = jnp.where(at_i, jnp.where(keep, cy1, 0.0), sel_y1)
        sel_x1 = jnp.where(at_i, jnp.where(keep, cx1, 0.0), sel_x1)
        sel_y2 = jnp.where(at_i, jnp.where(keep, cy2, 0.0), sel_y2)
        sel_x2 = jnp.where(at_i, jnp.where(keep, cx2, 0.0), sel_x2)

    s_out[:, :] = sel_s
    y1_out[:, :] = sel_y1
    x1_out[:, :] = sel_x1
    y2_out[:, :] = sel_y2
    x2_out[:, :] = sel_x2


def kernel(feat0, feat1, image_shape_t):
    del image_shape_t
    f0 = feat0.reshape(_N0, NUM_CLASSES + 5)
    f1 = feat1.reshape(_N1, NUM_CLASSES + 5)
    t = jnp.concatenate([f0, f1], axis=0)
    t = jnp.pad(t, ((0, _NP - _N), (0, 0)), constant_values=-1e9).T
    consts = jnp.asarray(_CONSTS)

    out_shapes = [jax.ShapeDtypeStruct((NUM_CLASSES, MAX_BOXES), jnp.float32)
                  for _ in range(5)]
    sel_s, sy1, sx1, sy2, sx2 = pl.pallas_call(
        _nms_body,
        out_shape=out_shapes,
    )(t, consts)

    boxes_ = jnp.stack([sy1, sx1, sy2, sx2], axis=-1).reshape(-1, 4)
    scores_ = sel_s.reshape(-1)
    classes_ = jnp.repeat(jnp.arange(NUM_CLASSES, dtype=jnp.int32), MAX_BOXES)
    return boxes_, scores_, classes_
